# baseline (speedup 1.0000x reference)
import jax
import jax.numpy as jnp
from jax.experimental import pallas as pl
from jax.experimental.pallas import tpu as pltpu

_BLK = 256


def _fused_body(x_ref, a_hbm, wself_ref, bself_ref, wnb_ref, bnb_ref, watt_ref,
                gamma_ref, beta_ref, out_hbm, m_scr, abuf, obuf, asem, osem):
    B, N, F = x_ref.shape
    nblk = N // _BLK

    def a_copy(i, slot):
        return pltpu.make_async_copy(
            a_hbm.at[pl.ds(i * _BLK, _BLK), :], abuf.at[slot], asem.at[slot])

    def o_copy(i, slot):
        return pltpu.make_async_copy(
            obuf.at[slot], out_hbm.at[:, pl.ds(i * _BLK, _BLK), :],
            osem.at[slot])

    a_copy(0, 0).start()
    a_copy(1, 1).start()

    w2 = watt_ref[1:2, :]
    es = []
    for b in range(B):
        x = x_ref[b]
        nb = (jnp.dot(x, wnb_ref[...], preferred_element_type=jnp.float32)
              + bnb_ref[...])
        att = jnp.sum(x * w2, axis=1, keepdims=True)
        e = jnp.exp(att - jnp.max(att))
        m_scr[:, b * F:(b + 1) * F] = e * nb
        es.append(e)
    es.append(jnp.zeros((N, F - B), dtype=jnp.float32))
    m_scr[:, B * F:] = jnp.concatenate(es, axis=1)

    for i in range(nblk):
        slot = i % 2
        a_copy(i, slot).wait()
        mm = jnp.dot(abuf[slot], m_scr[...],
                     preferred_element_type=jnp.float32)
        if i >= 2:
            o_copy(i - 2, slot).wait()
        for b in range(B):
            x_blk = x_ref[b, pl.ds(i * _BLK, _BLK), :]
            self_blk = (jnp.dot(x_blk, wself_ref[...],
                                preferred_element_type=jnp.float32)
                        + bself_ref[...])
            num = mm[:, b * F:(b + 1) * F]
            den = mm[:, B * F + b:B * F + b + 1]
            rec = jnp.where(den > 0, 1.0 / den, 0.0)
            comb = self_blk + num * rec
            mean = jnp.mean(comb, axis=1, keepdims=True)
            cent = comb - mean
            var = jnp.mean(cent * cent, axis=1, keepdims=True)
            rstd = jax.lax.rsqrt(var + 1e-5)
            obuf[slot, b] = jnp.maximum(
                (cent * rstd) * gamma_ref[...] + beta_ref[...], 0.0)
        o_copy(i, slot).start()
        if i + 2 < nblk:
            a_copy(i + 2, slot).start()
    o_copy(nblk - 2, (nblk - 2) % 2).wait()
    o_copy(nblk - 1, (nblk - 1) % 2).wait()


def kernel(node_features, adjacency_matrix, W_self, b_self, W_nb, b_nb,
           W_att, b_att, ln_gamma, ln_beta):
    B, N, F = node_features.shape
    watt2 = W_att.reshape(2, F)
    bself = b_self.reshape(1, F)
    bnb = b_nb.reshape(1, F)
    gamma = ln_gamma.reshape(1, F)
    beta = ln_beta.reshape(1, F)

    out = pl.pallas_call(
        _fused_body,
        in_specs=[
            pl.BlockSpec((B, N, F), lambda: (0, 0, 0)),
            pl.BlockSpec(memory_space=pltpu.MemorySpace.HBM),
            pl.BlockSpec((F, F), lambda: (0, 0)),
            pl.BlockSpec((1, F), lambda: (0, 0)),
            pl.BlockSpec((F, F), lambda: (0, 0)),
            pl.BlockSpec((1, F), lambda: (0, 0)),
            pl.BlockSpec((2, F), lambda: (0, 0)),
            pl.BlockSpec((1, F), lambda: (0, 0)),
            pl.BlockSpec((1, F), lambda: (0, 0)),
        ],
        out_specs=pl.BlockSpec(memory_space=pltpu.MemorySpace.HBM),
        out_shape=jax.ShapeDtypeStruct((B, N, F), jnp.float32),
        scratch_shapes=[
            pltpu.VMEM((N, (B + 1) * F), jnp.float32),
            pltpu.VMEM((2, _BLK, N), jnp.float32),
            pltpu.VMEM((2, B, _BLK, F), jnp.float32),
            pltpu.SemaphoreType.DMA((2,)),
            pltpu.SemaphoreType.DMA((2,)),
        ],
    )(node_features, adjacency_matrix, W_self, bself, W_nb, bnb, watt2,
      gamma, beta)
    return out
